# baseline (speedup 1.0000x reference)
import functools

import jax
import jax.numpy as jnp
from jax import lax
from jax.experimental import pallas as pl
from jax.experimental.pallas import tpu as pltpu
from jax.experimental.pallas import tpu_sc as plsc

BATCH = 4096
HIDDEN = 64
BASKET = 20
NW = 32
BPW = BATCH // NW
CHUNK = 64
GROWS = CHUNK * BASKET
NGD = GROWS // 128


def _sc_body(sidx_hbm, u_hbm, a_hbm, item_hbm, usr_hbm, q_out, k_out,
             sidx_v, uidx_v, aidx_v, rows_v, urows_v, arows_v, q_v,
             gsem, usem, asem):
    wid = lax.axis_index("s") * 2 + lax.axis_index("c")

    def chunk_body(c, carry):
        base = wid * BPW + c * CHUNK
        pltpu.sync_copy(sidx_hbm.at[pl.ds(base * BASKET, GROWS)], sidx_v)
        pltpu.sync_copy(u_hbm.at[pl.ds(base, CHUNK)], uidx_v)
        pltpu.sync_copy(a_hbm.at[pl.ds(base, CHUNK)], aidx_v)
        cps = []
        for r in range(NGD):
            cps.append(pltpu.async_copy(
                item_hbm.at[sidx_v.at[pl.ds(r * 128, 128)]],
                rows_v.at[pl.ds(r * 128, 128)], gsem))
        cu = pltpu.async_copy(usr_hbm.at[uidx_v], urows_v, usem)
        ca = pltpu.async_copy(item_hbm.at[aidx_v], arows_v, asem)
        for cp in cps:
            cp.wait()
        cu.wait()
        ca.wait()

        def row_body(b, carry2):
            for hs in range(HIDDEN // 16):
                sl = pl.ds(hs * 16, 16)
                acc = urows_v[b, sl]
                for j in range(BASKET):
                    acc = acc + rows_v[b * BASKET + j, sl]
                q_v[b, sl] = acc
            return carry2

        lax.fori_loop(0, CHUNK, row_body, 0)
        pltpu.sync_copy(q_v, q_out.at[pl.ds(base, CHUNK)])
        pltpu.sync_copy(arows_v, k_out.at[pl.ds(base, CHUNK)])
        return carry

    lax.fori_loop(0, BPW // CHUNK, chunk_body, 0)


_sc_embed = functools.partial(
    pl.kernel,
    out_type=(
        jax.ShapeDtypeStruct((BATCH, HIDDEN), jnp.float32),
        jax.ShapeDtypeStruct((BATCH, HIDDEN), jnp.float32),
    ),
    mesh=plsc.VectorSubcoreMesh(core_axis_name="c", subcore_axis_name="s"),
    compiler_params=pltpu.CompilerParams(use_tc_tiling_on_sc=False),
    scratch_types=[
        pltpu.VMEM((GROWS,), jnp.int32),
        pltpu.VMEM((CHUNK,), jnp.int32),
        pltpu.VMEM((CHUNK,), jnp.int32),
        pltpu.VMEM((GROWS, HIDDEN), jnp.float32),
        pltpu.VMEM((CHUNK, HIDDEN), jnp.float32),
        pltpu.VMEM((CHUNK, HIDDEN), jnp.float32),
        pltpu.VMEM((CHUNK, HIDDEN), jnp.float32),
        pltpu.SemaphoreType.DMA,
        pltpu.SemaphoreType.DMA,
        pltpu.SemaphoreType.DMA,
    ],
)(_sc_body)


def _mm_body(q_ref, k_ref, o_ref):
    o_ref[...] = lax.dot_general(
        q_ref[...], k_ref[...],
        dimension_numbers=(((1,), (1,)), ((), ())),
        preferred_element_type=jnp.float32)


def _logits(q, k):
    bm = bn = 512
    return pl.pallas_call(
        _mm_body,
        grid=(BATCH // bm, BATCH // bn),
        in_specs=[
            pl.BlockSpec((bm, HIDDEN), lambda i, j: (i, 0)),
            pl.BlockSpec((bn, HIDDEN), lambda i, j: (j, 0)),
        ],
        out_specs=pl.BlockSpec((bm, bn), lambda i, j: (i, j)),
        out_shape=jax.ShapeDtypeStruct((BATCH, BATCH), jnp.float32),
    )(q, k)


def kernel(U, S, A, B, item_embedding, usr_embedding):
    del B
    s_last = S[:, -1, :].astype(jnp.int32).reshape(BATCH * BASKET)
    q, k = _sc_embed(s_last, U.astype(jnp.int32), A.astype(jnp.int32),
                     item_embedding, usr_embedding)
    return _logits(q, k)


# speedup vs baseline: 1.0719x; 1.0719x over previous
import functools

import jax
import jax.numpy as jnp
from jax import lax
from jax.experimental import pallas as pl
from jax.experimental.pallas import tpu as pltpu
from jax.experimental.pallas import tpu_sc as plsc

BATCH = 4096
HIDDEN = 64
HPAD = 128
BASKET = 20
NW = 32
BPW = BATCH // NW
CHUNK = 32
GROWS = CHUNK * BASKET
NGD = GROWS // 128


def _sc_body(sidx_hbm, u_hbm, a_hbm, item_hbm, usr_hbm, q_out, k_out,
             sidx_v, uidx_v, aidx_v, rows_v, urows_v, arows_v, q_v,
             gsem, usem, asem):
    wid = lax.axis_index("s") * 2 + lax.axis_index("c")

    def chunk_body(c, carry):
        base = wid * BPW + c * CHUNK
        pltpu.sync_copy(sidx_hbm.at[pl.ds(base * BASKET, GROWS)], sidx_v)
        pltpu.sync_copy(u_hbm.at[pl.ds(base, CHUNK)], uidx_v)
        pltpu.sync_copy(a_hbm.at[pl.ds(base, CHUNK)], aidx_v)
        cps = []
        for r in range(NGD):
            cps.append(pltpu.async_copy(
                item_hbm.at[sidx_v.at[pl.ds(r * 128, 128)]],
                rows_v.at[pl.ds(r * 128, 128)], gsem))
        cu = pltpu.async_copy(usr_hbm.at[uidx_v], urows_v, usem)
        ca = pltpu.async_copy(item_hbm.at[aidx_v], arows_v, asem)
        for cp in cps:
            cp.wait()
        cu.wait()
        ca.wait()

        def row_body(b, carry2):
            for hs in range(HIDDEN // 16):
                sl = pl.ds(hs * 16, 16)
                acc = urows_v[b, sl]
                for j in range(BASKET):
                    acc = acc + rows_v[b * BASKET + j, sl]
                q_v[b, sl] = acc
            return carry2

        lax.fori_loop(0, CHUNK, row_body, 0)
        pltpu.sync_copy(q_v, q_out.at[pl.ds(base, CHUNK)])
        pltpu.sync_copy(arows_v, k_out.at[pl.ds(base, CHUNK)])
        return carry

    lax.fori_loop(0, BPW // CHUNK, chunk_body, 0)


_sc_embed = functools.partial(
    pl.kernel,
    out_type=(
        jax.ShapeDtypeStruct((BATCH, HIDDEN), jnp.float32),
        jax.ShapeDtypeStruct((BATCH, HPAD), jnp.float32),
    ),
    mesh=plsc.VectorSubcoreMesh(core_axis_name="c", subcore_axis_name="s"),
    compiler_params=pltpu.CompilerParams(use_tc_tiling_on_sc=False),
    scratch_types=[
        pltpu.VMEM((GROWS,), jnp.int32),
        pltpu.VMEM((CHUNK,), jnp.int32),
        pltpu.VMEM((CHUNK,), jnp.int32),
        pltpu.VMEM((GROWS, HPAD), jnp.float32),
        pltpu.VMEM((CHUNK, HPAD), jnp.float32),
        pltpu.VMEM((CHUNK, HPAD), jnp.float32),
        pltpu.VMEM((CHUNK, HIDDEN), jnp.float32),
        pltpu.SemaphoreType.DMA,
        pltpu.SemaphoreType.DMA,
        pltpu.SemaphoreType.DMA,
    ],
)(_sc_body)


def _mm_body(q_ref, k_ref, o_ref):
    o_ref[...] = lax.dot_general(
        q_ref[...], k_ref[:, :HIDDEN],
        dimension_numbers=(((1,), (1,)), ((), ())),
        preferred_element_type=jnp.float32)


def _logits(q, k):
    bm = bn = 512
    return pl.pallas_call(
        _mm_body,
        grid=(BATCH // bm, BATCH // bn),
        in_specs=[
            pl.BlockSpec((bm, HIDDEN), lambda i, j: (i, 0)),
            pl.BlockSpec((bn, HPAD), lambda i, j: (j, 0)),
        ],
        out_specs=pl.BlockSpec((bm, bn), lambda i, j: (i, j)),
        out_shape=jax.ShapeDtypeStruct((BATCH, BATCH), jnp.float32),
    )(q, k)


def kernel(U, S, A, B, item_embedding, usr_embedding):
    del B
    item128 = jnp.pad(item_embedding, ((0, 0), (0, HPAD - HIDDEN)))
    usr128 = jnp.pad(usr_embedding, ((0, 0), (0, HPAD - HIDDEN)))
    s_last = S[:, -1, :].astype(jnp.int32).reshape(BATCH * BASKET)
    q, k = _sc_embed(s_last, U.astype(jnp.int32), A.astype(jnp.int32),
                     item128, usr128)
    return _logits(q, k)
